# Initial kernel scaffold; baseline (speedup 1.0000x reference)
import jax
import jax.numpy as jnp
from jax.experimental import pallas as pl


def kernel(x, edge_index, W_neigh1, W_self1, b_self1, W_neigh2, W_self2, b_self2):
    raise NotImplementedError("write your pallas kernel here")


# speedup vs baseline: 3.1669x; 3.1669x over previous
import functools

import jax
import jax.numpy as jnp
from jax import lax
from jax.experimental import pallas as pl
from jax.experimental.pallas import tpu as pltpu
from jax.experimental.pallas import tpu_sc as plsc

N = 10000
D = 128
E = 320000

NUM_CORES = 2
NUM_SUBCORES = 16
NW = NUM_CORES * NUM_SUBCORES

N_PAD = 10240
ROWS_SEG = N_PAD // NUM_SUBCORES
DUMP = N_PAD
N_ACC = N_PAD + 8

BLK = 128
BURST = 8
EPW = 10240
E_PAD = NW * EPW
N_BURSTS = EPW // BLK // BURST


def _sc_agg_body(h_hbm, src_hbm, dst_hbm, zeros2d_hbm, zeros1d_hbm, ones_hbm,
                 sums_out, deg_out,
                 src_v, dst_v, rows_v, ones_v, acc_sh, deg_sh, sem):
    c = lax.axis_index("c")
    s = lax.axis_index("s")
    wid = s * NUM_CORES + c

    pltpu.sync_copy(zeros2d_hbm, acc_sh.at[pl.ds(s * ROWS_SEG, ROWS_SEG)])
    pltpu.sync_copy(zeros1d_hbm, deg_sh.at[pl.ds(s * ROWS_SEG, ROWS_SEG)])
    pltpu.sync_copy(ones_hbm, ones_v)
    plsc.subcore_barrier()

    blk0 = wid * (EPW // BLK)

    def burst(i, carry):
        b = blk0 + i * BURST
        pltpu.sync_copy(src_hbm.at[pl.ds(b, BURST)], src_v)
        pltpu.sync_copy(dst_hbm.at[pl.ds(b, BURST)], dst_v)
        for j in range(BURST):
            pltpu.async_copy(h_hbm.at[src_v.at[j]], rows_v, sem).wait()
            pltpu.sync_copy(rows_v, acc_sh.at[dst_v.at[j]], add=True)
            pltpu.sync_copy(ones_v, deg_sh.at[dst_v.at[j]], add=True)
        return carry

    lax.fori_loop(0, N_BURSTS, burst, 0)

    plsc.subcore_barrier()
    pltpu.sync_copy(acc_sh.at[pl.ds(s * ROWS_SEG, ROWS_SEG)],
                    sums_out.at[c].at[pl.ds(s * ROWS_SEG, ROWS_SEG)])
    pltpu.sync_copy(deg_sh.at[pl.ds(s * ROWS_SEG, ROWS_SEG)],
                    deg_out.at[c].at[pl.ds(s * ROWS_SEG, ROWS_SEG)])


_sc_agg = functools.partial(
    pl.kernel,
    out_type=(jax.ShapeDtypeStruct((NUM_CORES, N_PAD, D), jnp.float32),
              jax.ShapeDtypeStruct((NUM_CORES, N_PAD), jnp.float32)),
    mesh=plsc.VectorSubcoreMesh(core_axis_name="c", subcore_axis_name="s"),
    scratch_types=[
        pltpu.VMEM((BURST, BLK), jnp.int32),
        pltpu.VMEM((BURST, BLK), jnp.int32),
        pltpu.VMEM((BLK, D), jnp.float32),
        pltpu.VMEM((BLK,), jnp.float32),
        pltpu.VMEM_SHARED((N_ACC, D), jnp.float32),
        pltpu.VMEM_SHARED((N_ACC,), jnp.float32),
        pltpu.SemaphoreType.DMA,
    ],
)(_sc_agg_body)


RB = 1024


def _tc_layer_body(act, x_ref, s0_ref, s1_ref, d0_ref, d1_ref,
                   wn_ref, ws_ref, b_ref, o_ref):
    deg = jnp.maximum(d0_ref[...] + d1_ref[...], 1.0)
    summed = s0_ref[...] + s1_ref[...]
    neigh = jnp.dot(summed, wn_ref[...],
                    preferred_element_type=jnp.float32) / deg
    out = jnp.dot(x_ref[...], ws_ref[...],
                  preferred_element_type=jnp.float32) + b_ref[...] + neigh
    if act:
        out = jnp.maximum(out, 0.0)
    o_ref[...] = out


def _tc_layer(x, s0, s1, d0, d1, wn_t, ws_t, b2d, act):
    return pl.pallas_call(
        functools.partial(_tc_layer_body, act),
        grid=(N_PAD // RB,),
        in_specs=[
            pl.BlockSpec((RB, D), lambda i: (i, 0)),
            pl.BlockSpec((RB, D), lambda i: (i, 0)),
            pl.BlockSpec((RB, D), lambda i: (i, 0)),
            pl.BlockSpec((RB, 1), lambda i: (i, 0)),
            pl.BlockSpec((RB, 1), lambda i: (i, 0)),
            pl.BlockSpec((D, D), lambda i: (0, 0)),
            pl.BlockSpec((D, D), lambda i: (0, 0)),
            pl.BlockSpec((1, D), lambda i: (0, 0)),
        ],
        out_specs=pl.BlockSpec((RB, D), lambda i: (i, 0)),
        out_shape=jax.ShapeDtypeStruct((N_PAD, D), jnp.float32),
    )(x, s0, s1, d0, d1, wn_t, ws_t, b2d)


def kernel(x, edge_index, W_neigh1, W_self1, b_self1,
           W_neigh2, W_self2, b_self2):
    src = edge_index[0]
    dst = edge_index[1]
    pad_e = E_PAD - E
    src_p = jnp.concatenate(
        [src, jnp.zeros((pad_e,), jnp.int32)]).reshape(E_PAD // BLK, BLK)
    dst_p = jnp.concatenate(
        [dst, jnp.full((pad_e,), DUMP, jnp.int32)]).reshape(E_PAD // BLK, BLK)
    x_p = jnp.concatenate(
        [x, jnp.zeros((N_PAD - N, D), jnp.float32)], axis=0)
    zeros2d = jnp.zeros((ROWS_SEG, D), jnp.float32)
    zeros1d = jnp.zeros((ROWS_SEG,), jnp.float32)
    ones = jnp.ones((BLK,), jnp.float32)

    sums1, deg = _sc_agg(x_p, src_p, dst_p, zeros2d, zeros1d, ones)
    d0 = deg[0].reshape(N_PAD, 1)
    d1 = deg[1].reshape(N_PAD, 1)
    h = _tc_layer(x_p, sums1[0], sums1[1], d0, d1,
                  W_neigh1.T, W_self1.T, b_self1.reshape(1, D), True)
    sums2, _ = _sc_agg(h, src_p, dst_p, zeros2d, zeros1d, ones)
    out = _tc_layer(h, sums2[0], sums2[1], d0, d1,
                    W_neigh2.T, W_self2.T, b_self2.reshape(1, D), False)
    return out[:N]


# speedup vs baseline: 3.4546x; 1.0909x over previous
import functools

import jax
import jax.numpy as jnp
from jax import lax
from jax.experimental import pallas as pl
from jax.experimental.pallas import tpu as pltpu
from jax.experimental.pallas import tpu_sc as plsc

N = 10000
D = 128
E = 320000

NUM_CORES = 2
NUM_SUBCORES = 16
NW = NUM_CORES * NUM_SUBCORES

N_PAD = 10240
ROWS_SEG = N_PAD // NUM_SUBCORES
DUMP = N_PAD
N_ACC = N_PAD + 8

BLK = 128
BURST = 16
NBUF = 2
EPW = 10240
E_PAD = NW * EPW
N_BURSTS = EPW // BLK // BURST


def _make_sc_agg(want_deg):
    def body(h_hbm, src_hbm, dst_hbm, zeros2d_hbm, zeros1d_hbm, ones_hbm,
             sums_out, deg_out,
             src_v, dst_v, rows_v, ones_v, acc_sh, deg_sh,
             gsem, ssem, dsem):
        c = lax.axis_index("c")
        s = lax.axis_index("s")
        wid = s * NUM_CORES + c

        pltpu.sync_copy(zeros2d_hbm, acc_sh.at[pl.ds(s * ROWS_SEG, ROWS_SEG)])
        if want_deg:
            pltpu.sync_copy(zeros1d_hbm,
                            deg_sh.at[pl.ds(s * ROWS_SEG, ROWS_SEG)])
            pltpu.sync_copy(ones_hbm, ones_v)
        plsc.subcore_barrier()

        blk0 = wid * (EPW // BLK)

        def burst(b, carry):
            base = blk0 + b * BURST
            pltpu.sync_copy(src_hbm.at[pl.ds(base, BURST)], src_v)
            pltpu.sync_copy(dst_hbm.at[pl.ds(base, BURST)], dst_v)
            sd = {}
            dd = {}
            gd = {0: pltpu.async_copy(
                h_hbm.at[src_v.at[0]], rows_v.at[0], gsem)}
            for j in range(BURST):
                gd[j].wait()
                sd[j] = pltpu.async_copy(
                    rows_v.at[j % NBUF], acc_sh.at[dst_v.at[j]], ssem,
                    add=True)
                if want_deg:
                    dd[j] = pltpu.async_copy(
                        ones_v, deg_sh.at[dst_v.at[j]], dsem, add=True)
                if j >= 1:
                    sd[j - 1].wait()
                if j + 1 < BURST:
                    gd[j + 1] = pltpu.async_copy(
                        h_hbm.at[src_v.at[j + 1]], rows_v.at[(j + 1) % NBUF],
                        gsem)
            sd[BURST - 1].wait()
            if want_deg:
                for j in range(BURST):
                    dd[j].wait()
            return carry

        lax.fori_loop(0, N_BURSTS, burst, 0)

        plsc.subcore_barrier()
        pltpu.sync_copy(acc_sh.at[pl.ds(s * ROWS_SEG, ROWS_SEG)],
                        sums_out.at[c].at[pl.ds(s * ROWS_SEG, ROWS_SEG)])
        if want_deg:
            pltpu.sync_copy(deg_sh.at[pl.ds(s * ROWS_SEG, ROWS_SEG)],
                            deg_out.at[c].at[pl.ds(s * ROWS_SEG, ROWS_SEG)])

    return functools.partial(
        pl.kernel,
        out_type=(jax.ShapeDtypeStruct((NUM_CORES, N_PAD, D), jnp.float32),
                  jax.ShapeDtypeStruct((NUM_CORES, N_PAD), jnp.float32)),
        mesh=plsc.VectorSubcoreMesh(core_axis_name="c", subcore_axis_name="s"),
        scratch_types=[
            pltpu.VMEM((BURST, BLK), jnp.int32),
            pltpu.VMEM((BURST, BLK), jnp.int32),
            pltpu.VMEM((NBUF, BLK, D), jnp.float32),
            pltpu.VMEM((BLK,), jnp.float32),
            pltpu.VMEM_SHARED((N_ACC, D), jnp.float32),
            pltpu.VMEM_SHARED((N_ACC,), jnp.float32),
            pltpu.SemaphoreType.DMA,
            pltpu.SemaphoreType.DMA,
            pltpu.SemaphoreType.DMA,
        ],
    )(body)


_sc_agg = _make_sc_agg(True)


RB = 1024


def _tc_layer_body(act, x_ref, s0_ref, s1_ref, d0_ref, d1_ref,
                   wn_ref, ws_ref, b_ref, o_ref):
    deg = jnp.maximum(d0_ref[...] + d1_ref[...], 1.0)
    summed = s0_ref[...] + s1_ref[...]
    neigh = jnp.dot(summed, wn_ref[...],
                    preferred_element_type=jnp.float32) / deg
    out = jnp.dot(x_ref[...], ws_ref[...],
                  preferred_element_type=jnp.float32) + b_ref[...] + neigh
    if act:
        out = jnp.maximum(out, 0.0)
    o_ref[...] = out


def _tc_layer(x, s0, s1, d0, d1, wn_t, ws_t, b2d, act):
    return pl.pallas_call(
        functools.partial(_tc_layer_body, act),
        grid=(N_PAD // RB,),
        in_specs=[
            pl.BlockSpec((RB, D), lambda i: (i, 0)),
            pl.BlockSpec((RB, D), lambda i: (i, 0)),
            pl.BlockSpec((RB, D), lambda i: (i, 0)),
            pl.BlockSpec((RB, 1), lambda i: (i, 0)),
            pl.BlockSpec((RB, 1), lambda i: (i, 0)),
            pl.BlockSpec((D, D), lambda i: (0, 0)),
            pl.BlockSpec((D, D), lambda i: (0, 0)),
            pl.BlockSpec((1, D), lambda i: (0, 0)),
        ],
        out_specs=pl.BlockSpec((RB, D), lambda i: (i, 0)),
        out_shape=jax.ShapeDtypeStruct((N_PAD, D), jnp.float32),
    )(x, s0, s1, d0, d1, wn_t, ws_t, b2d)


def kernel(x, edge_index, W_neigh1, W_self1, b_self1,
           W_neigh2, W_self2, b_self2):
    src = edge_index[0]
    dst = edge_index[1]
    pad_e = E_PAD - E
    src_p = jnp.concatenate(
        [src, jnp.zeros((pad_e,), jnp.int32)]).reshape(E_PAD // BLK, BLK)
    dst_p = jnp.concatenate(
        [dst, jnp.full((pad_e,), DUMP, jnp.int32)]).reshape(E_PAD // BLK, BLK)
    x_p = jnp.concatenate(
        [x, jnp.zeros((N_PAD - N, D), jnp.float32)], axis=0)
    zeros2d = jnp.zeros((ROWS_SEG, D), jnp.float32)
    zeros1d = jnp.zeros((ROWS_SEG,), jnp.float32)
    ones = jnp.ones((BLK,), jnp.float32)

    sums1, deg = _sc_agg(x_p, src_p, dst_p, zeros2d, zeros1d, ones)
    d0 = deg[0].reshape(N_PAD, 1)
    d1 = deg[1].reshape(N_PAD, 1)
    h = _tc_layer(x_p, sums1[0], sums1[1], d0, d1,
                  W_neigh1.T, W_self1.T, b_self1.reshape(1, D), True)
    sums2, _ = _sc_agg(h, src_p, dst_p, zeros2d, zeros1d, ones)
    out = _tc_layer(h, sums2[0], sums2[1], d0, d1,
                    W_neigh2.T, W_self2.T, b_self2.reshape(1, D), False)
    return out[:N]


# speedup vs baseline: 3.4582x; 1.0010x over previous
import functools

import jax
import jax.numpy as jnp
from jax import lax
from jax.experimental import pallas as pl
from jax.experimental.pallas import tpu as pltpu
from jax.experimental.pallas import tpu_sc as plsc

N = 10000
D = 128
E = 320000

NUM_CORES = 2
NUM_SUBCORES = 16
NW = NUM_CORES * NUM_SUBCORES

N_PAD = 10240
ROWS_SEG = N_PAD // NUM_SUBCORES
N_DUMP = 128
N_ACC = N_PAD + N_DUMP

BLK = 128
BURST = 16
NBUF = 2
EPW = 10240
E_PAD = NW * EPW
N_BURSTS = EPW // BLK // BURST


def _make_sc_agg(want_deg):
    def body(h_hbm, src_hbm, dst_hbm, zeros2d_hbm, zeros1d_hbm, ones_hbm,
             sums_out, deg_out,
             src_v, dst_v, rows_v, ones_v, acc_sh, deg_sh,
             gsem, ssem, dsem):
        c = lax.axis_index("c")
        s = lax.axis_index("s")
        wid = s * NUM_CORES + c

        pltpu.sync_copy(zeros2d_hbm, acc_sh.at[pl.ds(s * ROWS_SEG, ROWS_SEG)])
        if want_deg:
            pltpu.sync_copy(zeros1d_hbm,
                            deg_sh.at[pl.ds(s * ROWS_SEG, ROWS_SEG)])
            pltpu.sync_copy(ones_hbm, ones_v)
        plsc.subcore_barrier()

        blk0 = wid * (EPW // BLK)

        def burst(b, carry):
            base = blk0 + b * BURST
            pltpu.sync_copy(src_hbm.at[pl.ds(base, BURST)], src_v)
            pltpu.sync_copy(dst_hbm.at[pl.ds(base, BURST)], dst_v)
            sd = {}
            dd = {}
            gd = {0: pltpu.async_copy(
                h_hbm.at[src_v.at[0]], rows_v.at[0], gsem)}
            for j in range(BURST):
                gd[j].wait()
                sd[j] = pltpu.async_copy(
                    rows_v.at[j % NBUF], acc_sh.at[dst_v.at[j]], ssem,
                    add=True)
                if want_deg:
                    dd[j] = pltpu.async_copy(
                        ones_v, deg_sh.at[dst_v.at[j]], dsem, add=True)
                if j >= 1:
                    sd[j - 1].wait()
                if j + 1 < BURST:
                    gd[j + 1] = pltpu.async_copy(
                        h_hbm.at[src_v.at[j + 1]], rows_v.at[(j + 1) % NBUF],
                        gsem)
            sd[BURST - 1].wait()
            if want_deg:
                for j in range(BURST):
                    dd[j].wait()
            return carry

        lax.fori_loop(0, N_BURSTS, burst, 0)

        plsc.subcore_barrier()
        pltpu.sync_copy(acc_sh.at[pl.ds(s * ROWS_SEG, ROWS_SEG)],
                        sums_out.at[c].at[pl.ds(s * ROWS_SEG, ROWS_SEG)])
        if want_deg:
            pltpu.sync_copy(deg_sh.at[pl.ds(s * ROWS_SEG, ROWS_SEG)],
                            deg_out.at[c].at[pl.ds(s * ROWS_SEG, ROWS_SEG)])

    return functools.partial(
        pl.kernel,
        out_type=(jax.ShapeDtypeStruct((NUM_CORES, N_PAD, D), jnp.float32),
                  jax.ShapeDtypeStruct((NUM_CORES, N_PAD), jnp.float32)),
        mesh=plsc.VectorSubcoreMesh(core_axis_name="c", subcore_axis_name="s"),
        scratch_types=[
            pltpu.VMEM((BURST, BLK), jnp.int32),
            pltpu.VMEM((BURST, BLK), jnp.int32),
            pltpu.VMEM((NBUF, BLK, D), jnp.float32),
            pltpu.VMEM((BLK,), jnp.float32),
            pltpu.VMEM_SHARED((N_ACC, D), jnp.float32),
            pltpu.VMEM_SHARED((N_ACC,), jnp.float32),
            pltpu.SemaphoreType.DMA,
            pltpu.SemaphoreType.DMA,
            pltpu.SemaphoreType.DMA,
        ],
    )(body)


_sc_agg = _make_sc_agg(True)


RB = 1024


def _tc_layer_body(act, x_ref, s0_ref, s1_ref, d0_ref, d1_ref,
                   wn_ref, ws_ref, b_ref, o_ref):
    deg = jnp.maximum(d0_ref[...] + d1_ref[...], 1.0)
    summed = s0_ref[...] + s1_ref[...]
    neigh = jnp.dot(summed, wn_ref[...],
                    preferred_element_type=jnp.float32) / deg
    out = jnp.dot(x_ref[...], ws_ref[...],
                  preferred_element_type=jnp.float32) + b_ref[...] + neigh
    if act:
        out = jnp.maximum(out, 0.0)
    o_ref[...] = out


def _tc_layer(x, s0, s1, d0, d1, wn_t, ws_t, b2d, act):
    return pl.pallas_call(
        functools.partial(_tc_layer_body, act),
        grid=(N_PAD // RB,),
        in_specs=[
            pl.BlockSpec((RB, D), lambda i: (i, 0)),
            pl.BlockSpec((RB, D), lambda i: (i, 0)),
            pl.BlockSpec((RB, D), lambda i: (i, 0)),
            pl.BlockSpec((RB, 1), lambda i: (i, 0)),
            pl.BlockSpec((RB, 1), lambda i: (i, 0)),
            pl.BlockSpec((D, D), lambda i: (0, 0)),
            pl.BlockSpec((D, D), lambda i: (0, 0)),
            pl.BlockSpec((1, D), lambda i: (0, 0)),
        ],
        out_specs=pl.BlockSpec((RB, D), lambda i: (i, 0)),
        out_shape=jax.ShapeDtypeStruct((N_PAD, D), jnp.float32),
    )(x, s0, s1, d0, d1, wn_t, ws_t, b2d)


def kernel(x, edge_index, W_neigh1, W_self1, b_self1,
           W_neigh2, W_self2, b_self2):
    src = edge_index[0]
    dst = edge_index[1]
    pad_e = E_PAD - E
    src_p = jnp.concatenate(
        [src, jnp.zeros((pad_e,), jnp.int32)]).reshape(E_PAD // BLK, BLK)
    dump_idx = N_PAD + (jnp.arange(pad_e, dtype=jnp.int32) % N_DUMP)
    dst_p = jnp.concatenate([dst, dump_idx]).reshape(E_PAD // BLK, BLK)
    x_p = jnp.concatenate(
        [x, jnp.zeros((N_PAD - N, D), jnp.float32)], axis=0)
    zeros2d = jnp.zeros((ROWS_SEG, D), jnp.float32)
    zeros1d = jnp.zeros((ROWS_SEG,), jnp.float32)
    ones = jnp.ones((BLK,), jnp.float32)

    sums1, deg = _sc_agg(x_p, src_p, dst_p, zeros2d, zeros1d, ones)
    d0 = deg[0].reshape(N_PAD, 1)
    d1 = deg[1].reshape(N_PAD, 1)
    h = _tc_layer(x_p, sums1[0], sums1[1], d0, d1,
                  W_neigh1.T, W_self1.T, b_self1.reshape(1, D), True)
    sums2, _ = _sc_agg(h, src_p, dst_p, zeros2d, zeros1d, ones)
    out = _tc_layer(h, sums2[0], sums2[1], d0, d1,
                    W_neigh2.T, W_self2.T, b_self2.reshape(1, D), False)
    return out[:N]
